# Initial kernel scaffold; baseline (speedup 1.0000x reference)
import functools

import jax
import jax.numpy as jnp
from jax import lax
from jax.experimental import pallas as pl
from jax.experimental.pallas import tpu as pltpu
from jax.experimental.pallas import tpu_sc as plsc

NC, NS = 2, 16
NW = NC * NS
DC = 128
WIN = 80


def _sc_mesh():
    return plsc.VectorSubcoreMesh(core_axis_name="c", subcore_axis_name="s")


def _deg_body(dst1, dst2, ones_h, zeros_h, out, idx_v, ones_v, zbuf_v, acc):
    c = lax.axis_index("c")
    s = lax.axis_index("s")
    wid = c * NS + s
    e = dst1.shape[0]
    epw = e // NW
    nwin = epw // WIN
    n = out.shape[2]
    nmain = (n // (NS * 8)) * 8
    rem = n - NS * nmain
    pltpu.sync_copy(ones_h, ones_v)
    pltpu.sync_copy(zeros_h, zbuf_v)
    off = s * nmain
    for br in range(2):
        pltpu.sync_copy(zbuf_v.at[pl.ds(0, nmain)], acc.at[br, pl.ds(off, nmain)])
        if rem:
            @pl.when(s == 0)
            def _():
                pltpu.sync_copy(zbuf_v.at[pl.ds(0, rem)],
                                acc.at[br, pl.ds(NS * nmain, rem)])
    plsc.subcore_barrier()
    for br, dref in ((0, dst1), (1, dst2)):
        base = wid * epw

        def body(j, carry):
            pltpu.sync_copy(dref.at[pl.ds(base + j * WIN, WIN)], idx_v)
            pltpu.sync_copy(ones_v, acc.at[br].at[idx_v], add=True)
            return carry

        lax.fori_loop(0, nwin, body, 0)
    plsc.subcore_barrier()
    for br in range(2):
        pltpu.sync_copy(acc.at[br, pl.ds(off, nmain)],
                        out.at[c, br, pl.ds(off, nmain)])
        if rem:
            @pl.when(s == 0)
            def _():
                pltpu.sync_copy(acc.at[br, pl.ds(NS * nmain, rem)],
                                out.at[c, br, pl.ds(NS * nmain, rem)])


def _sc_deg(dst1, dst2, n):
    ones_h = jnp.ones((WIN,), jnp.float32)
    zeros_h = jnp.zeros((640,), jnp.float32)
    k = pl.kernel(
        _deg_body,
        out_type=jax.ShapeDtypeStruct((NC, 2, n), jnp.float32),
        mesh=_sc_mesh(),
        scratch_types=[
            pltpu.VMEM((WIN,), jnp.int32),
            pltpu.VMEM((WIN,), jnp.float32),
            pltpu.VMEM((640,), jnp.float32),
            pltpu.VMEM_SHARED((2, n), jnp.float32),
        ],
    )
    return k(dst1, dst2, ones_h, zeros_h)


def _agg_body(hp, srcr, dstr, zrows_h, out, sidx, didx, rows, zrows, acc, sem):
    c = lax.axis_index("c")
    s = lax.axis_index("s")
    wid = c * NS + s
    nb = hp.shape[0]
    n = hp.shape[1]
    e = srcr.shape[0]
    epw = e // NW
    nwin = epw // WIN
    base = wid * epw
    nmain = (n // (NS * 8)) * 8
    rem = n - NS * nmain
    roff = s * nmain
    pltpu.sync_copy(zrows_h, zrows)
    for b in range(nb):
        pltpu.sync_copy(zrows, acc.at[pl.ds(roff, nmain)])
        if rem:
            @pl.when(s == 0)
            def _():
                pltpu.sync_copy(zrows.at[pl.ds(0, rem)],
                                acc.at[pl.ds(NS * nmain, rem)])
        plsc.subcore_barrier()
        table = hp.at[b]

        def body(j, carry):
            pltpu.sync_copy(srcr.at[pl.ds(base + j * WIN, WIN)], sidx)
            pltpu.sync_copy(dstr.at[pl.ds(base + j * WIN, WIN)], didx)
            pltpu.async_copy(table.at[sidx], rows, sem).wait()
            pltpu.sync_copy(rows, acc.at[didx], add=True)
            return carry

        lax.fori_loop(0, nwin, body, 0)
        plsc.subcore_barrier()
        pltpu.sync_copy(acc.at[pl.ds(roff, nmain)],
                        out.at[c, b, pl.ds(roff, nmain)])
        if rem:
            @pl.when(s == 0)
            def _():
                pltpu.sync_copy(acc.at[pl.ds(NS * nmain, rem)],
                                out.at[c, b, pl.ds(NS * nmain, rem)])
        plsc.subcore_barrier()


def _sc_agg(hp3, src, dst):
    nb, n, dc = hp3.shape
    nmain = (n // (NS * 8)) * 8
    zrows_h = jnp.zeros((nmain, dc), jnp.float32)
    k = pl.kernel(
        _agg_body,
        out_type=jax.ShapeDtypeStruct((NC, nb, n, dc), jnp.float32),
        mesh=_sc_mesh(),
        scratch_types=[
            pltpu.VMEM((WIN,), jnp.int32),
            pltpu.VMEM((WIN,), jnp.int32),
            pltpu.VMEM((WIN, dc), jnp.float32),
            pltpu.VMEM((nmain, dc), jnp.float32),
            pltpu.VMEM_SHARED((n, dc), jnp.float32),
            pltpu.SemaphoreType.DMA,
        ],
    )
    return k(hp3, src, dst, zrows_h)


def _pre_body(x1, x2, degp, xs1, xs2, d1o, d2o):
    dp = degp[...]
    i1 = lax.rsqrt(dp[0, 0] + dp[1, 0] + 1.0)
    i2 = lax.rsqrt(dp[0, 1] + dp[1, 1] + 1.0)
    xs1[...] = x1[...] * i1
    xs2[...] = x2[...] * i2
    d1o[...] = i1
    d2o[...] = i2


def _tc_pre(x1, x2, degp):
    n, din = x1.shape
    return pl.pallas_call(
        _pre_body,
        out_shape=[
            jax.ShapeDtypeStruct((n, din), jnp.float32),
            jax.ShapeDtypeStruct((n, din), jnp.float32),
            jax.ShapeDtypeStruct((n, 1), jnp.float32),
            jax.ShapeDtypeStruct((n, 1), jnp.float32),
        ],
    )(x1, x2, degp)


def _mm_body(xs, w, out):
    out[...] = jnp.dot(xs[...], w[...],
                       preferred_element_type=jnp.float32)[None]


def _tc_mm(xs, w):
    n, kdim = xs.shape
    d = w.shape[1]
    nb = d // DC
    return pl.pallas_call(
        _mm_body,
        grid=(nb,),
        in_specs=[
            pl.BlockSpec((n, kdim), lambda b: (0, 0)),
            pl.BlockSpec((kdim, DC), lambda b: (0, b)),
        ],
        out_specs=pl.BlockSpec((1, n, DC), lambda b: (b, 0, 0)),
        out_shape=jax.ShapeDtypeStruct((nb, n, DC), jnp.float32),
    )(xs, w)


def _post_body(part, hp, dinv, bias, g, bt, out, *, last):
    p = part[...]
    dv = dinv[...]
    z = (p[0, 0] + p[1, 0] + hp[...][0]) * dv + bias[...]
    m = jnp.mean(z, axis=0, keepdims=True)
    v = jnp.mean((z - m) ** 2, axis=0, keepdims=True)
    a = jnp.maximum(g[...] * (z - m) * lax.rsqrt(v + 1e-5) + bt[...], 0.0)
    out[...] = a if last else a * dv


def _tc_post(part, hp3, dinv, bias, g, bt, last):
    _, nb, n, dc = part.shape
    d = nb * dc
    return pl.pallas_call(
        functools.partial(_post_body, last=last),
        grid=(nb,),
        in_specs=[
            pl.BlockSpec((2, 1, n, dc), lambda b: (0, b, 0, 0)),
            pl.BlockSpec((1, n, dc), lambda b: (b, 0, 0)),
            pl.BlockSpec((n, 1), lambda b: (0, 0)),
            pl.BlockSpec((1, dc), lambda b: (0, b)),
            pl.BlockSpec((1, dc), lambda b: (0, b)),
            pl.BlockSpec((1, dc), lambda b: (0, b)),
        ],
        out_specs=pl.BlockSpec((n, dc), lambda b: (0, b)),
        out_shape=jax.ShapeDtypeStruct((n, d), jnp.float32),
    )(part, hp3, dinv, bias.reshape(1, d), g.reshape(1, d), bt.reshape(1, d))


def _pool_body(act, batch, f1w, f1b, f2w, f2b, out, pooled):
    a = act[...]
    bt = batch[...]
    ng = pooled.shape[0]

    def body(b, carry):
        mask = (bt == b).astype(jnp.float32)
        pooled[pl.ds(b, 1), :] = jnp.max(a * mask, axis=0, keepdims=True)
        return carry

    lax.fori_loop(0, ng, body, 0)
    p = pooled[...]
    f = jnp.maximum(
        jnp.dot(p, f1w[...], preferred_element_type=jnp.float32) + f1b[...],
        0.0)
    out[...] = jnp.dot(f, f2w[...],
                       preferred_element_type=jnp.float32) + f2b[...]


def _tc_pool(act, batch2d, f1w, f1b, f2w, f2b, ng):
    d = act.shape[1]
    dmid = f1w.shape[1]
    dout = f2w.shape[1]
    return pl.pallas_call(
        _pool_body,
        out_shape=jax.ShapeDtypeStruct((ng, dout), jnp.float32),
        scratch_shapes=[pltpu.VMEM((ng, d), jnp.float32)],
    )(act, batch2d, f1w, f1b.reshape(1, dmid), f2w, f2b.reshape(1, dout))


def _head_body(xd1, xd2, cell, cw, cb, m1w, m1b, m2w, m2b, m3w, m3b, out):
    cl = jnp.maximum(
        jnp.dot(cell[...], cw[...], preferred_element_type=jnp.float32)
        + cb[...], 0.0)
    w = m1w[...]
    h = (jnp.dot(xd1[...], w[0:256], preferred_element_type=jnp.float32)
         + jnp.dot(xd2[...], w[256:512], preferred_element_type=jnp.float32)
         + jnp.dot(cl, w[512:768], preferred_element_type=jnp.float32)
         + m1b[...])
    h = jnp.maximum(h, 0.0)
    h = jnp.maximum(
        jnp.dot(h, m2w[...], preferred_element_type=jnp.float32) + m2b[...],
        0.0)
    o = jnp.dot(h, m3w[...], preferred_element_type=jnp.float32) + m3b[...]
    out[...] = 200.0 / (1.0 + jnp.exp(-o)) - 100.0


def _tc_head(xd1, xd2, cell, p):
    b = cell.shape[0]
    return pl.pallas_call(
        _head_body,
        out_shape=jax.ShapeDtypeStruct((b, 1), jnp.float32),
    )(xd1, xd2, cell, p["c_W"], p["c_b"].reshape(1, -1),
      p["m1W"], p["m1b"].reshape(1, -1), p["m2W"], p["m2b"].reshape(1, -1),
      p["m3W"], p["m3b"].reshape(1, -1))


def _gcn_branch(xs, dinv, src, dst, batch2d, p, pre, ng):
    a = xs
    for l, last in ((1, False), (2, False), (3, True)):
        hp3 = _tc_mm(a, p[pre + f"W{l}"])
        part = _sc_agg(hp3, src, dst)
        a = _tc_post(part, hp3, dinv, p[pre + f"b{l}"],
                     p[pre + f"g{l}"], p[pre + f"bt{l}"], last)
    return _tc_pool(a, batch2d, p[pre + "f1W"], p[pre + "f1b"],
                    p[pre + "f2W"], p[pre + "f2b"], ng)


def kernel(x1, edge_index1, x2, edge_index2, cellline, batch, params):
    p = params
    n = x1.shape[0]
    ng = cellline.shape[0]
    src1, dst1 = edge_index1[0], edge_index1[1]
    src2, dst2 = edge_index2[0], edge_index2[1]
    degp = _sc_deg(dst1, dst2, n).reshape(NC, 2, n, 1)
    xs1, xs2, dinv1, dinv2 = _tc_pre(x1, x2, degp)
    batch2d = batch.reshape(n, 1)
    xd1 = _gcn_branch(xs1, dinv1, src1, dst1, batch2d, p, "d1_", ng)
    xd2 = _gcn_branch(xs2, dinv2, src2, dst2, batch2d, p, "d2_", ng)
    return _tc_head(xd1, xd2, cellline, p)


# speedup vs baseline: 5.5697x; 5.5697x over previous
import functools

import jax
import jax.numpy as jnp
from jax import lax
from jax.experimental import pallas as pl
from jax.experimental.pallas import tpu as pltpu
from jax.experimental.pallas import tpu_sc as plsc

NC, NS = 2, 16
NW = NC * NS
DC = 128
WIN = 80


def _sc_mesh():
    return plsc.VectorSubcoreMesh(core_axis_name="c", subcore_axis_name="s")


def _deg_body(dst1, dst2, ones_h, zeros_h, out, idx_v, ones_v, zbuf_v,
              acc1, acc2):
    c = lax.axis_index("c")
    s = lax.axis_index("s")
    wid = c * NS + s
    e = dst1.shape[0]
    epw = e // NW
    nwin = epw // WIN
    n = acc1.shape[0]
    nmain = (n // (NS * 8)) * 8
    rem = n - NS * nmain
    pltpu.sync_copy(ones_h, ones_v)
    pltpu.sync_copy(zeros_h, zbuf_v)
    off = s * nmain
    for acc in (acc1, acc2):
        pltpu.sync_copy(zbuf_v.at[pl.ds(0, nmain)], acc.at[pl.ds(off, nmain)])
        if rem:
            @pl.when(s == 0)
            def _():
                pltpu.sync_copy(zbuf_v.at[pl.ds(0, rem)],
                                acc.at[pl.ds(NS * nmain, rem)])
    plsc.subcore_barrier()
    for acc, dref in ((acc1, dst1), (acc2, dst2)):
        base = wid * epw

        def body(j, carry):
            pltpu.sync_copy(dref.at[pl.ds(base + j * WIN, WIN)], idx_v)
            pltpu.sync_copy(ones_v, acc.at[idx_v], add=True)
            return carry

        lax.fori_loop(0, nwin, body, 0)
    plsc.subcore_barrier()
    for br, acc in ((0, acc1), (1, acc2)):
        obase = (c * 2 + br) * n
        pltpu.sync_copy(acc.at[pl.ds(off, nmain)], zbuf_v.at[pl.ds(0, nmain)])
        pltpu.sync_copy(zbuf_v.at[pl.ds(0, nmain)],
                        out.at[pl.ds(obase + off, nmain)])
        if rem:
            @pl.when(s == 0)
            def _():
                pltpu.sync_copy(acc.at[pl.ds(NS * nmain, rem)],
                                zbuf_v.at[pl.ds(0, rem)])
                pltpu.sync_copy(zbuf_v.at[pl.ds(0, rem)],
                                out.at[pl.ds(obase + NS * nmain, rem)])


def _sc_deg(dst1, dst2, n):
    ones_h = jnp.ones((WIN,), jnp.float32)
    zeros_h = jnp.zeros((640,), jnp.float32)
    k = pl.kernel(
        _deg_body,
        out_type=jax.ShapeDtypeStruct((NC * 2 * n,), jnp.float32),
        mesh=_sc_mesh(),
        scratch_types=[
            pltpu.VMEM((WIN,), jnp.int32),
            pltpu.VMEM((WIN,), jnp.float32),
            pltpu.VMEM((640,), jnp.float32),
            pltpu.VMEM_SHARED((n,), jnp.float32),
            pltpu.VMEM_SHARED((n,), jnp.float32),
        ],
    )
    return k(dst1, dst2, ones_h, zeros_h)


def _chunk(nmain):
    for d in range(128, 0, -8):
        if nmain % d == 0:
            return d
    return 8


def _agg_body(hp, srcr, dstr, zrows_h, out, sidx, didx, rows, zrows, obuf,
              acc, sem):
    c = lax.axis_index("c")
    s = lax.axis_index("s")
    wid = c * NS + s
    nb = hp.shape[0]
    n = hp.shape[1]
    e = srcr.shape[0]
    epw = e // NW
    nwin = epw // WIN
    base = wid * epw
    nmain = (n // (NS * 8)) * 8
    rem = n - NS * nmain
    roff = s * nmain
    ch = zrows.shape[0]
    nch = nmain // ch
    pltpu.sync_copy(zrows_h, zrows)
    for b in range(nb):
        for q in range(nch):
            pltpu.sync_copy(zrows, acc.at[pl.ds(roff + q * ch, ch)])
        if rem:
            @pl.when(s == 0)
            def _():
                pltpu.sync_copy(zrows.at[pl.ds(0, rem)],
                                acc.at[pl.ds(NS * nmain, rem)])
        plsc.subcore_barrier()
        table = hp.at[b]

        def body(j, carry):
            pltpu.sync_copy(srcr.at[pl.ds(base + j * WIN, WIN)], sidx)
            pltpu.sync_copy(dstr.at[pl.ds(base + j * WIN, WIN)], didx)
            pltpu.async_copy(table.at[sidx], rows, sem).wait()
            pltpu.sync_copy(rows, acc.at[didx], add=True)
            return carry

        lax.fori_loop(0, nwin, body, 0)
        plsc.subcore_barrier()
        for q in range(nch):
            pltpu.sync_copy(acc.at[pl.ds(roff + q * ch, ch)], obuf)
            pltpu.sync_copy(obuf, out.at[c, b, pl.ds(roff + q * ch, ch)])
        if rem:
            @pl.when(s == 0)
            def _():
                pltpu.sync_copy(acc.at[pl.ds(NS * nmain, rem)],
                                obuf.at[pl.ds(0, rem)])
                pltpu.sync_copy(obuf.at[pl.ds(0, rem)],
                                out.at[c, b, pl.ds(NS * nmain, rem)])
        plsc.subcore_barrier()


def _sc_agg(hp3, src, dst):
    nb, n, dc = hp3.shape
    nmain = (n // (NS * 8)) * 8
    ch = _chunk(nmain)
    zrows_h = jnp.zeros((ch, dc), jnp.float32)
    k = pl.kernel(
        _agg_body,
        out_type=jax.ShapeDtypeStruct((NC, nb, n, dc), jnp.float32),
        mesh=_sc_mesh(),
        scratch_types=[
            pltpu.VMEM((WIN,), jnp.int32),
            pltpu.VMEM((WIN,), jnp.int32),
            pltpu.VMEM((WIN, dc), jnp.float32),
            pltpu.VMEM((ch, dc), jnp.float32),
            pltpu.VMEM((ch, dc), jnp.float32),
            pltpu.VMEM_SHARED((n, dc), jnp.float32),
            pltpu.SemaphoreType.DMA,
        ],
    )
    return k(hp3, src, dst, zrows_h)


def _pre_body(x1, x2, degp, xs1, xs2, d1o, d2o):
    dp = degp[...]
    i1 = lax.rsqrt(dp[0, 0] + dp[1, 0] + 1.0)
    i2 = lax.rsqrt(dp[0, 1] + dp[1, 1] + 1.0)
    xs1[...] = x1[...] * i1
    xs2[...] = x2[...] * i2
    d1o[...] = i1
    d2o[...] = i2


def _tc_pre(x1, x2, degp):
    n, din = x1.shape
    return pl.pallas_call(
        _pre_body,
        out_shape=[
            jax.ShapeDtypeStruct((n, din), jnp.float32),
            jax.ShapeDtypeStruct((n, din), jnp.float32),
            jax.ShapeDtypeStruct((n, 1), jnp.float32),
            jax.ShapeDtypeStruct((n, 1), jnp.float32),
        ],
    )(x1, x2, degp)


def _mm_body(xs, w, out):
    out[...] = jnp.dot(xs[...], w[...],
                       preferred_element_type=jnp.float32, precision=lax.Precision.HIGHEST)[None]


def _tc_mm(xs, w):
    n, kdim = xs.shape
    d = w.shape[1]
    nb = d // DC
    return pl.pallas_call(
        _mm_body,
        grid=(nb,),
        in_specs=[
            pl.BlockSpec((n, kdim), lambda b: (0, 0)),
            pl.BlockSpec((kdim, DC), lambda b: (0, b)),
        ],
        out_specs=pl.BlockSpec((1, n, DC), lambda b: (b, 0, 0)),
        out_shape=jax.ShapeDtypeStruct((nb, n, DC), jnp.float32),
    )(xs, w)


def _post_body(part, hp, dinv, bias, g, bt, out, *, last):
    p = part[...]
    dv = dinv[...]
    z = (p[0, 0] + p[1, 0] + hp[...][0]) * dv + bias[...]
    m = jnp.mean(z, axis=0, keepdims=True)
    v = jnp.mean((z - m) ** 2, axis=0, keepdims=True)
    a = jnp.maximum(g[...] * (z - m) * lax.rsqrt(v + 1e-5) + bt[...], 0.0)
    out[...] = a if last else a * dv


def _tc_post(part, hp3, dinv, bias, g, bt, last):
    _, nb, n, dc = part.shape
    d = nb * dc
    return pl.pallas_call(
        functools.partial(_post_body, last=last),
        grid=(nb,),
        in_specs=[
            pl.BlockSpec((2, 1, n, dc), lambda b: (0, b, 0, 0)),
            pl.BlockSpec((1, n, dc), lambda b: (b, 0, 0)),
            pl.BlockSpec((n, 1), lambda b: (0, 0)),
            pl.BlockSpec((1, dc), lambda b: (0, b)),
            pl.BlockSpec((1, dc), lambda b: (0, b)),
            pl.BlockSpec((1, dc), lambda b: (0, b)),
        ],
        out_specs=pl.BlockSpec((n, dc), lambda b: (0, b)),
        out_shape=jax.ShapeDtypeStruct((n, d), jnp.float32),
    )(part, hp3, dinv, bias.reshape(1, d), g.reshape(1, d), bt.reshape(1, d))


def _pool_body(act, batch, f1w, f1b, f2w, f2b, out, pooled):
    a = act[...]
    bt = batch[...]
    ng = pooled.shape[0]

    def body(b, carry):
        mask = (bt == b).astype(jnp.float32)
        pooled[pl.ds(b, 1), :] = jnp.max(a * mask, axis=0, keepdims=True)
        return carry

    lax.fori_loop(0, ng, body, 0)
    p = pooled[...]
    f = jnp.maximum(
        jnp.dot(p, f1w[...], preferred_element_type=jnp.float32, precision=lax.Precision.HIGHEST) + f1b[...],
        0.0)
    out[...] = jnp.dot(f, f2w[...],
                       preferred_element_type=jnp.float32, precision=lax.Precision.HIGHEST) + f2b[...]


def _tc_pool(act, batch2d, f1w, f1b, f2w, f2b, ng):
    d = act.shape[1]
    dmid = f1w.shape[1]
    dout = f2w.shape[1]
    return pl.pallas_call(
        _pool_body,
        out_shape=jax.ShapeDtypeStruct((ng, dout), jnp.float32),
        scratch_shapes=[pltpu.VMEM((ng, d), jnp.float32)],
    )(act, batch2d, f1w, f1b.reshape(1, dmid), f2w, f2b.reshape(1, dout))


def _head_body(xd1, xd2, cell, cw, cb, m1w, m1b, m2w, m2b, m3w, m3b, out):
    cl = jnp.maximum(
        jnp.dot(cell[...], cw[...], preferred_element_type=jnp.float32, precision=lax.Precision.HIGHEST)
        + cb[...], 0.0)
    w = m1w[...]
    h = (jnp.dot(xd1[...], w[0:256], preferred_element_type=jnp.float32, precision=lax.Precision.HIGHEST)
         + jnp.dot(xd2[...], w[256:512], preferred_element_type=jnp.float32, precision=lax.Precision.HIGHEST)
         + jnp.dot(cl, w[512:768], preferred_element_type=jnp.float32, precision=lax.Precision.HIGHEST)
         + m1b[...])
    h = jnp.maximum(h, 0.0)
    h = jnp.maximum(
        jnp.dot(h, m2w[...], preferred_element_type=jnp.float32, precision=lax.Precision.HIGHEST) + m2b[...],
        0.0)
    o = jnp.dot(h, m3w[...], preferred_element_type=jnp.float32, precision=lax.Precision.HIGHEST) + m3b[...]
    out[...] = 200.0 / (1.0 + jnp.exp(-o)) - 100.0


def _tc_head(xd1, xd2, cell, p):
    b = cell.shape[0]
    return pl.pallas_call(
        _head_body,
        out_shape=jax.ShapeDtypeStruct((b, 1), jnp.float32),
    )(xd1, xd2, cell, p["c_W"], p["c_b"].reshape(1, -1),
      p["m1W"], p["m1b"].reshape(1, -1), p["m2W"], p["m2b"].reshape(1, -1),
      p["m3W"], p["m3b"].reshape(1, -1))


def _gcn_branch(xs, dinv, src, dst, batch2d, p, pre, ng):
    a = xs
    for l, last in ((1, False), (2, False), (3, True)):
        hp3 = _tc_mm(a, p[pre + f"W{l}"])
        part = _sc_agg(hp3, src, dst)
        a = _tc_post(part, hp3, dinv, p[pre + f"b{l}"],
                     p[pre + f"g{l}"], p[pre + f"bt{l}"], last)
    return _tc_pool(a, batch2d, p[pre + "f1W"], p[pre + "f1b"],
                    p[pre + "f2W"], p[pre + "f2b"], ng)


def kernel(x1, edge_index1, x2, edge_index2, cellline, batch, params):
    p = params
    n = x1.shape[0]
    ng = cellline.shape[0]
    src1, dst1 = edge_index1[0], edge_index1[1]
    src2, dst2 = edge_index2[0], edge_index2[1]
    degp = _sc_deg(dst1, dst2, n).reshape(NC, 2, n, 1)
    xs1, xs2, dinv1, dinv2 = _tc_pre(x1, x2, degp)
    batch2d = batch.reshape(n, 1)
    xd1 = _gcn_branch(xs1, dinv1, src1, dst1, batch2d, p, "d1_", ng)
    xd2 = _gcn_branch(xs2, dinv2, src2, dst2, batch2d, p, "d2_", ng)
    return _tc_head(xd1, xd2, cellline, p)


# speedup vs baseline: 10.6938x; 1.9200x over previous
import functools

import jax
import jax.numpy as jnp
from jax import lax
from jax.experimental import pallas as pl
from jax.experimental.pallas import tpu as pltpu
from jax.experimental.pallas import tpu_sc as plsc

NC, NS = 2, 16
NW = NC * NS
DC = 128
WIN = 40


def _sc_mesh():
    return plsc.VectorSubcoreMesh(core_axis_name="c", subcore_axis_name="s")


def _deg_body(dst1, dst2, ones_h, zeros_h, out, idx_v, ones_v, zbuf_v,
              acc1, acc2):
    c = lax.axis_index("c")
    s = lax.axis_index("s")
    wid = c * NS + s
    e = dst1.shape[0]
    epw = e // NW
    nwin = epw // WIN
    n = acc1.shape[0]
    nmain = (n // (NS * 8)) * 8
    rem = n - NS * nmain
    pltpu.sync_copy(ones_h, ones_v)
    pltpu.sync_copy(zeros_h, zbuf_v)
    off = s * nmain
    for acc in (acc1, acc2):
        pltpu.sync_copy(zbuf_v.at[pl.ds(0, nmain)], acc.at[pl.ds(off, nmain)])
        if rem:
            @pl.when(s == 0)
            def _():
                pltpu.sync_copy(zbuf_v.at[pl.ds(0, rem)],
                                acc.at[pl.ds(NS * nmain, rem)])
    plsc.subcore_barrier()
    for acc, dref in ((acc1, dst1), (acc2, dst2)):
        base = wid * epw

        def body(j, carry):
            pltpu.sync_copy(dref.at[pl.ds(base + j * WIN, WIN)], idx_v)
            pltpu.sync_copy(ones_v, acc.at[idx_v], add=True)
            return carry

        lax.fori_loop(0, nwin, body, 0)
    plsc.subcore_barrier()
    for br, acc in ((0, acc1), (1, acc2)):
        obase = (c * 2 + br) * n
        pltpu.sync_copy(acc.at[pl.ds(off, nmain)], zbuf_v.at[pl.ds(0, nmain)])
        pltpu.sync_copy(zbuf_v.at[pl.ds(0, nmain)],
                        out.at[pl.ds(obase + off, nmain)])
        if rem:
            @pl.when(s == 0)
            def _():
                pltpu.sync_copy(acc.at[pl.ds(NS * nmain, rem)],
                                zbuf_v.at[pl.ds(0, rem)])
                pltpu.sync_copy(zbuf_v.at[pl.ds(0, rem)],
                                out.at[pl.ds(obase + NS * nmain, rem)])


def _sc_deg(dst1, dst2, n):
    ones_h = jnp.ones((WIN,), jnp.float32)
    zeros_h = jnp.zeros((640,), jnp.float32)
    k = pl.kernel(
        _deg_body,
        out_type=jax.ShapeDtypeStruct((NC * 2 * n,), jnp.float32),
        mesh=_sc_mesh(),
        scratch_types=[
            pltpu.VMEM((WIN,), jnp.int32),
            pltpu.VMEM((WIN,), jnp.float32),
            pltpu.VMEM((640,), jnp.float32),
            pltpu.VMEM_SHARED((n,), jnp.float32),
            pltpu.VMEM_SHARED((n,), jnp.float32),
        ],
    )
    return k(dst1, dst2, ones_h, zeros_h)


def _chunk(nmain):
    for d in range(128, 0, -8):
        if nmain % d == 0:
            return d
    return 8


NBUF = 5


def _agg_body(hp, srcr, dstr, zrows_h, out,
              x0, x1, x2, x3, x4, d0, d1, d2, d3, d4,
              r0, r1, r2, r3, r4, zrows, obufa, obufb, acc,
              g0, g1, g2, g3, g4, s0, s1, s2, s3, s4,
              i0, i1, i2, i3, i4, zsem, oa, ob):
    c = lax.axis_index("c")
    s = lax.axis_index("s")
    wid = c * NS + s
    nb = hp.shape[0]
    n = hp.shape[1]
    e = srcr.shape[0]
    epw = e // NW
    nwin = epw // WIN
    base = wid * epw
    nmain = (n // (NS * 8)) * 8
    rem = n - NS * nmain
    roff = s * nmain
    ch = zrows.shape[0]
    nch = nmain // ch
    rows = (r0, r1, r2, r3, r4)
    sidxb = (x0, x1, x2, x3, x4)
    didxb = (d0, d1, d2, d3, d4)
    gsem = (g0, g1, g2, g3, g4)
    ssem = (s0, s1, s2, s3, s4)
    isem = (i0, i1, i2, i3, i4)
    obufs = (obufa, obufb)
    osems = (oa, ob)
    pltpu.sync_copy(zrows_h, zrows)

    def fetch_idx(j, k):
        pltpu.async_copy(srcr.at[pl.ds(base + j * WIN, WIN)], sidxb[k],
                         isem[k])
        pltpu.async_copy(dstr.at[pl.ds(base + j * WIN, WIN)], didxb[k],
                         isem[k])

    def wait_idx(k):
        pltpu.make_async_copy(srcr.at[pl.ds(base, WIN)], sidxb[k],
                              isem[k]).wait()
        pltpu.make_async_copy(dstr.at[pl.ds(base, WIN)], didxb[k],
                              isem[k]).wait()

    for b in range(nb):
        for q in range(nch):
            pltpu.async_copy(zrows, acc.at[pl.ds(roff + q * ch, ch)], zsem)
        for q in range(nch):
            pltpu.make_async_copy(
                zrows, acc.at[pl.ds(roff + q * ch, ch)], zsem).wait()
        if rem:
            @pl.when(s == 0)
            def _():
                pltpu.sync_copy(zrows.at[pl.ds(0, rem)],
                                acc.at[pl.ds(NS * nmain, rem)])
        plsc.subcore_barrier()
        table = hp.at[b]
        for k in range(NBUF):
            fetch_idx(k, k)
        for k in range(NBUF):
            wait_idx(k)
            pltpu.async_copy(table.at[sidxb[k]], rows[k], gsem[k])

        def grp(g, carry):
            for k in range(NBUF):
                pltpu.make_async_copy(table.at[sidxb[k]], rows[k],
                                      gsem[k]).wait()
                pltpu.async_copy(rows[k], acc.at[didxb[k]], ssem[k],
                                 add=True)
            for k in range(NBUF):
                j = g * NBUF + k
                pltpu.make_async_copy(rows[k], acc.at[didxb[k]],
                                      ssem[k]).wait()

                @pl.when(j + NBUF < nwin)
                def _():
                    fetch_idx(j + NBUF, k)
            for k in range(NBUF):
                j = g * NBUF + k

                @pl.when(j + NBUF < nwin)
                def _():
                    wait_idx(k)
                    pltpu.async_copy(table.at[sidxb[k]], rows[k], gsem[k])
            return carry

        lax.fori_loop(0, nwin // NBUF, grp, 0)
        plsc.subcore_barrier()
        for q in range(nch):
            ob_ = obufs[q % 2]
            os_ = osems[q % 2]
            if q >= 2:
                pltpu.make_async_copy(
                    ob_, out.at[c, b, pl.ds(roff + (q - 2) * ch, ch)],
                    os_).wait()
            pltpu.sync_copy(acc.at[pl.ds(roff + q * ch, ch)], ob_)
            pltpu.async_copy(ob_, out.at[c, b, pl.ds(roff + q * ch, ch)], os_)
        for q in range(max(nch - 2, 0), nch):
            pltpu.make_async_copy(
                obufs[q % 2], out.at[c, b, pl.ds(roff + q * ch, ch)],
                osems[q % 2]).wait()
        if rem:
            @pl.when(s == 0)
            def _():
                pltpu.sync_copy(acc.at[pl.ds(NS * nmain, rem)],
                                obufa.at[pl.ds(0, rem)])
                pltpu.sync_copy(obufa.at[pl.ds(0, rem)],
                                out.at[c, b, pl.ds(NS * nmain, rem)])
        plsc.subcore_barrier()


def _sc_agg(hp3, src, dst):
    nb, n, dc = hp3.shape
    ch = 16
    zrows_h = jnp.zeros((ch, dc), jnp.float32)
    k = pl.kernel(
        _agg_body,
        out_type=jax.ShapeDtypeStruct((NC, nb, n, dc), jnp.float32),
        mesh=_sc_mesh(),
        scratch_types=(
            [pltpu.VMEM((WIN,), jnp.int32) for _ in range(2 * NBUF)]
            + [pltpu.VMEM((WIN, dc), jnp.float32) for _ in range(NBUF)]
            + [pltpu.VMEM((ch, dc), jnp.float32),
               pltpu.VMEM((ch, dc), jnp.float32),
               pltpu.VMEM((ch, dc), jnp.float32),
               pltpu.VMEM_SHARED((n, dc), jnp.float32)]
            + [pltpu.SemaphoreType.DMA for _ in range(3 * NBUF + 3)]
        ),
    )
    return k(hp3, src, dst, zrows_h)


def _pre_body(x1, x2, degp, xs1, xs2, d1o, d2o):
    dp = degp[...]
    i1 = lax.rsqrt(dp[0, 0] + dp[1, 0] + 1.0)
    i2 = lax.rsqrt(dp[0, 1] + dp[1, 1] + 1.0)
    xs1[...] = x1[...] * i1
    xs2[...] = x2[...] * i2
    d1o[...] = i1
    d2o[...] = i2


def _tc_pre(x1, x2, degp):
    n, din = x1.shape
    return pl.pallas_call(
        _pre_body,
        out_shape=[
            jax.ShapeDtypeStruct((n, din), jnp.float32),
            jax.ShapeDtypeStruct((n, din), jnp.float32),
            jax.ShapeDtypeStruct((n, 1), jnp.float32),
            jax.ShapeDtypeStruct((n, 1), jnp.float32),
        ],
    )(x1, x2, degp)


def _mm_body(xs, w, out):
    out[...] = jnp.dot(xs[...], w[...],
                       preferred_element_type=jnp.float32, precision=lax.Precision.HIGHEST)[None]


def _tc_mm(xs, w):
    n, kdim = xs.shape
    d = w.shape[1]
    nb = d // DC
    return pl.pallas_call(
        _mm_body,
        grid=(nb,),
        in_specs=[
            pl.BlockSpec((n, kdim), lambda b: (0, 0)),
            pl.BlockSpec((kdim, DC), lambda b: (0, b)),
        ],
        out_specs=pl.BlockSpec((1, n, DC), lambda b: (b, 0, 0)),
        out_shape=jax.ShapeDtypeStruct((nb, n, DC), jnp.float32),
    )(xs, w)


def _post_body(part, hp, dinv, bias, g, bt, out, *, last):
    p = part[...]
    dv = dinv[...]
    z = (p[0, 0] + p[1, 0] + hp[...][0]) * dv + bias[...]
    m = jnp.mean(z, axis=0, keepdims=True)
    v = jnp.mean((z - m) ** 2, axis=0, keepdims=True)
    a = jnp.maximum(g[...] * (z - m) * lax.rsqrt(v + 1e-5) + bt[...], 0.0)
    out[...] = a if last else a * dv


def _tc_post(part, hp3, dinv, bias, g, bt, last):
    _, nb, n, dc = part.shape
    d = nb * dc
    return pl.pallas_call(
        functools.partial(_post_body, last=last),
        grid=(nb,),
        in_specs=[
            pl.BlockSpec((2, 1, n, dc), lambda b: (0, b, 0, 0)),
            pl.BlockSpec((1, n, dc), lambda b: (b, 0, 0)),
            pl.BlockSpec((n, 1), lambda b: (0, 0)),
            pl.BlockSpec((1, dc), lambda b: (0, b)),
            pl.BlockSpec((1, dc), lambda b: (0, b)),
            pl.BlockSpec((1, dc), lambda b: (0, b)),
        ],
        out_specs=pl.BlockSpec((n, dc), lambda b: (0, b)),
        out_shape=jax.ShapeDtypeStruct((n, d), jnp.float32),
    )(part, hp3, dinv, bias.reshape(1, d), g.reshape(1, d), bt.reshape(1, d))


def _pool_body(act, batch, f1w, f1b, f2w, f2b, out, pooled):
    a = act[...]
    bt = batch[...]
    ng = pooled.shape[0]

    def body(b, carry):
        mask = (bt == b).astype(jnp.float32)
        pooled[pl.ds(b, 1), :] = jnp.max(a * mask, axis=0, keepdims=True)
        return carry

    lax.fori_loop(0, ng, body, 0)
    p = pooled[...]
    f = jnp.maximum(
        jnp.dot(p, f1w[...], preferred_element_type=jnp.float32, precision=lax.Precision.HIGHEST) + f1b[...],
        0.0)
    out[...] = jnp.dot(f, f2w[...],
                       preferred_element_type=jnp.float32, precision=lax.Precision.HIGHEST) + f2b[...]


def _tc_pool(act, batch2d, f1w, f1b, f2w, f2b, ng):
    d = act.shape[1]
    dmid = f1w.shape[1]
    dout = f2w.shape[1]
    return pl.pallas_call(
        _pool_body,
        out_shape=jax.ShapeDtypeStruct((ng, dout), jnp.float32),
        scratch_shapes=[pltpu.VMEM((ng, d), jnp.float32)],
    )(act, batch2d, f1w, f1b.reshape(1, dmid), f2w, f2b.reshape(1, dout))


def _head_body(xd1, xd2, cell, cw, cb, m1w, m1b, m2w, m2b, m3w, m3b, out):
    cl = jnp.maximum(
        jnp.dot(cell[...], cw[...], preferred_element_type=jnp.float32, precision=lax.Precision.HIGHEST)
        + cb[...], 0.0)
    w = m1w[...]
    h = (jnp.dot(xd1[...], w[0:256], preferred_element_type=jnp.float32, precision=lax.Precision.HIGHEST)
         + jnp.dot(xd2[...], w[256:512], preferred_element_type=jnp.float32, precision=lax.Precision.HIGHEST)
         + jnp.dot(cl, w[512:768], preferred_element_type=jnp.float32, precision=lax.Precision.HIGHEST)
         + m1b[...])
    h = jnp.maximum(h, 0.0)
    h = jnp.maximum(
        jnp.dot(h, m2w[...], preferred_element_type=jnp.float32, precision=lax.Precision.HIGHEST) + m2b[...],
        0.0)
    o = jnp.dot(h, m3w[...], preferred_element_type=jnp.float32, precision=lax.Precision.HIGHEST) + m3b[...]
    out[...] = 200.0 / (1.0 + jnp.exp(-o)) - 100.0


def _tc_head(xd1, xd2, cell, p):
    b = cell.shape[0]
    return pl.pallas_call(
        _head_body,
        out_shape=jax.ShapeDtypeStruct((b, 1), jnp.float32),
    )(xd1, xd2, cell, p["c_W"], p["c_b"].reshape(1, -1),
      p["m1W"], p["m1b"].reshape(1, -1), p["m2W"], p["m2b"].reshape(1, -1),
      p["m3W"], p["m3b"].reshape(1, -1))


def _gcn_branch(xs, dinv, src, dst, batch2d, p, pre, ng):
    a = xs
    for l, last in ((1, False), (2, False), (3, True)):
        hp3 = _tc_mm(a, p[pre + f"W{l}"])
        part = _sc_agg(hp3, src, dst)
        a = _tc_post(part, hp3, dinv, p[pre + f"b{l}"],
                     p[pre + f"g{l}"], p[pre + f"bt{l}"], last)
    return _tc_pool(a, batch2d, p[pre + "f1W"], p[pre + "f1b"],
                    p[pre + "f2W"], p[pre + "f2b"], ng)


def kernel(x1, edge_index1, x2, edge_index2, cellline, batch, params):
    p = params
    n = x1.shape[0]
    ng = cellline.shape[0]
    src1, dst1 = edge_index1[0], edge_index1[1]
    src2, dst2 = edge_index2[0], edge_index2[1]
    degp = _sc_deg(dst1, dst2, n).reshape(NC, 2, n, 1)
    xs1, xs2, dinv1, dinv2 = _tc_pre(x1, x2, degp)
    batch2d = batch.reshape(n, 1)
    xd1 = _gcn_branch(xs1, dinv1, src1, dst1, batch2d, p, "d1_", ng)
    xd2 = _gcn_branch(xs2, dinv2, src2, dst2, batch2d, p, "d2_", ng)
    return _tc_head(xd1, xd2, cellline, p)


# speedup vs baseline: 11.8112x; 1.1045x over previous
import functools

import jax
import jax.numpy as jnp
from jax import lax
from jax.experimental import pallas as pl
from jax.experimental.pallas import tpu as pltpu
from jax.experimental.pallas import tpu_sc as plsc

NC, NS = 2, 16
NW = NC * NS
DC = 128
WIN = 40


def _sc_mesh():
    return plsc.VectorSubcoreMesh(core_axis_name="c", subcore_axis_name="s")


WDEG = 80
DDEPTH = 4


def _deg_body(dst1, dst2, ones_h, zeros_h, out, i0, i1, i2, i3, ones_v,
              zbuf_v, acc1, acc2, q0, q1, q2, q3, t0, t1, t2, t3):
    c = lax.axis_index("c")
    s = lax.axis_index("s")
    wid = c * NS + s
    e = dst1.shape[0]
    epw = e // NW
    nwin = epw // WDEG
    n = acc1.shape[0]
    nmain = (n // (NS * 8)) * 8
    rem = n - NS * nmain
    idxs = (i0, i1, i2, i3)
    isem = (q0, q1, q2, q3)
    ssem = (t0, t1, t2, t3)
    pltpu.sync_copy(ones_h, ones_v)
    pltpu.sync_copy(zeros_h, zbuf_v)
    off = s * nmain
    for acc in (acc1, acc2):
        pltpu.sync_copy(zbuf_v.at[pl.ds(0, nmain)], acc.at[pl.ds(off, nmain)])
        if rem:
            @pl.when(s == 0)
            def _():
                pltpu.sync_copy(zbuf_v.at[pl.ds(0, rem)],
                                acc.at[pl.ds(NS * nmain, rem)])
    plsc.subcore_barrier()
    for acc, dref in ((acc1, dst1), (acc2, dst2)):
        base = wid * epw
        for k in range(DDEPTH):
            pltpu.async_copy(dref.at[pl.ds(base + k * WDEG, WDEG)], idxs[k],
                             isem[k])

        def grp(g, carry):
            for k in range(DDEPTH):
                j = g * DDEPTH + k

                @pl.when(j < nwin)
                def _():
                    pltpu.make_async_copy(dref.at[pl.ds(base, WDEG)],
                                          idxs[k], isem[k]).wait()
                    pltpu.async_copy(ones_v, acc.at[idxs[k]], ssem[k],
                                     add=True)
            for k in range(DDEPTH):
                j = g * DDEPTH + k

                @pl.when(j < nwin)
                def _():
                    pltpu.make_async_copy(ones_v, acc.at[idxs[k]],
                                          ssem[k]).wait()

                @pl.when(j + DDEPTH < nwin)
                def _():
                    pltpu.async_copy(
                        dref.at[pl.ds(base + (j + DDEPTH) * WDEG, WDEG)],
                        idxs[k], isem[k])
            return carry

        lax.fori_loop(0, (nwin + DDEPTH - 1) // DDEPTH, grp, 0)
    plsc.subcore_barrier()
    for br, acc in ((0, acc1), (1, acc2)):
        obase = (c * 2 + br) * n
        pltpu.sync_copy(acc.at[pl.ds(off, nmain)], zbuf_v.at[pl.ds(0, nmain)])
        pltpu.sync_copy(zbuf_v.at[pl.ds(0, nmain)],
                        out.at[pl.ds(obase + off, nmain)])
        if rem:
            @pl.when(s == 0)
            def _():
                pltpu.sync_copy(acc.at[pl.ds(NS * nmain, rem)],
                                zbuf_v.at[pl.ds(0, rem)])
                pltpu.sync_copy(zbuf_v.at[pl.ds(0, rem)],
                                out.at[pl.ds(obase + NS * nmain, rem)])


def _sc_deg(dst1, dst2, n):
    ones_h = jnp.ones((WDEG,), jnp.float32)
    zeros_h = jnp.zeros((640,), jnp.float32)
    k = pl.kernel(
        _deg_body,
        out_type=jax.ShapeDtypeStruct((NC * 2 * n,), jnp.float32),
        mesh=_sc_mesh(),
        scratch_types=(
            [pltpu.VMEM((WDEG,), jnp.int32) for _ in range(DDEPTH)]
            + [pltpu.VMEM((WDEG,), jnp.float32),
               pltpu.VMEM((640,), jnp.float32),
               pltpu.VMEM_SHARED((n,), jnp.float32),
               pltpu.VMEM_SHARED((n,), jnp.float32)]
            + [pltpu.SemaphoreType.DMA for _ in range(2 * DDEPTH)]
        ),
    )
    return k(dst1, dst2, ones_h, zeros_h)


def _chunk(nmain):
    for d in range(128, 0, -8):
        if nmain % d == 0:
            return d
    return 8


NBUF = 5


def _agg_body(hp, srcr, dstr, zrows_h, out,
              x0, x1, x2, x3, x4, d0, d1, d2, d3, d4,
              r0, r1, r2, r3, r4, zrows, obufa, obufb, acc,
              g0, g1, g2, g3, g4, s0, s1, s2, s3, s4,
              i0, i1, i2, i3, i4, zsem, oa, ob):
    c = lax.axis_index("c")
    s = lax.axis_index("s")
    wid = c * NS + s
    nb = hp.shape[0]
    n = hp.shape[1]
    e = srcr.shape[0]
    epw = e // NW
    nwin = epw // WIN
    base = wid * epw
    nmain = (n // (NS * 8)) * 8
    rem = n - NS * nmain
    roff = s * nmain
    ch = zrows.shape[0]
    nch = nmain // ch
    rows = (r0, r1, r2, r3, r4)
    sidxb = (x0, x1, x2, x3, x4)
    didxb = (d0, d1, d2, d3, d4)
    gsem = (g0, g1, g2, g3, g4)
    ssem = (s0, s1, s2, s3, s4)
    isem = (i0, i1, i2, i3, i4)
    obufs = (obufa, obufb)
    osems = (oa, ob)
    pltpu.sync_copy(zrows_h, zrows)

    def fetch_idx(j, k):
        pltpu.async_copy(srcr.at[pl.ds(base + j * WIN, WIN)], sidxb[k],
                         isem[k])
        pltpu.async_copy(dstr.at[pl.ds(base + j * WIN, WIN)], didxb[k],
                         isem[k])

    def wait_idx(k):
        pltpu.make_async_copy(srcr.at[pl.ds(base, WIN)], sidxb[k],
                              isem[k]).wait()
        pltpu.make_async_copy(dstr.at[pl.ds(base, WIN)], didxb[k],
                              isem[k]).wait()

    for b in range(nb):
        for q in range(nch):
            pltpu.async_copy(zrows, acc.at[pl.ds(roff + q * ch, ch)], zsem)
        for q in range(nch):
            pltpu.make_async_copy(
                zrows, acc.at[pl.ds(roff + q * ch, ch)], zsem).wait()
        if rem:
            @pl.when(s == 0)
            def _():
                pltpu.sync_copy(zrows.at[pl.ds(0, rem)],
                                acc.at[pl.ds(NS * nmain, rem)])
        plsc.subcore_barrier()
        table = hp.at[b]
        for k in range(NBUF):
            fetch_idx(k, k)
        for k in range(NBUF):
            wait_idx(k)
            pltpu.async_copy(table.at[sidxb[k]], rows[k], gsem[k])

        def grp(g, carry):
            for k in range(NBUF):
                pltpu.make_async_copy(table.at[sidxb[k]], rows[k],
                                      gsem[k]).wait()
                pltpu.async_copy(rows[k], acc.at[didxb[k]], ssem[k],
                                 add=True)
            for k in range(NBUF):
                j = g * NBUF + k
                pltpu.make_async_copy(rows[k], acc.at[didxb[k]],
                                      ssem[k]).wait()

                @pl.when(j + NBUF < nwin)
                def _():
                    fetch_idx(j + NBUF, k)
            for k in range(NBUF):
                j = g * NBUF + k

                @pl.when(j + NBUF < nwin)
                def _():
                    wait_idx(k)
                    pltpu.async_copy(table.at[sidxb[k]], rows[k], gsem[k])
            return carry

        lax.fori_loop(0, nwin // NBUF, grp, 0)
        plsc.subcore_barrier()
        for q in range(nch):
            ob_ = obufs[q % 2]
            os_ = osems[q % 2]
            if q >= 2:
                pltpu.make_async_copy(
                    ob_, out.at[c, b, pl.ds(roff + (q - 2) * ch, ch)],
                    os_).wait()
            pltpu.sync_copy(acc.at[pl.ds(roff + q * ch, ch)], ob_)
            pltpu.async_copy(ob_, out.at[c, b, pl.ds(roff + q * ch, ch)], os_)
        for q in range(max(nch - 2, 0), nch):
            pltpu.make_async_copy(
                obufs[q % 2], out.at[c, b, pl.ds(roff + q * ch, ch)],
                osems[q % 2]).wait()
        if rem:
            @pl.when(s == 0)
            def _():
                pltpu.sync_copy(acc.at[pl.ds(NS * nmain, rem)],
                                obufa.at[pl.ds(0, rem)])
                pltpu.sync_copy(obufa.at[pl.ds(0, rem)],
                                out.at[c, b, pl.ds(NS * nmain, rem)])
        plsc.subcore_barrier()


def _sc_agg(hp3, src, dst):
    nb, n, dc = hp3.shape
    ch = 24
    zrows_h = jnp.zeros((ch, dc), jnp.float32)
    k = pl.kernel(
        _agg_body,
        out_type=jax.ShapeDtypeStruct((NC, nb, n, dc), jnp.float32),
        mesh=_sc_mesh(),
        scratch_types=(
            [pltpu.VMEM((WIN,), jnp.int32) for _ in range(2 * NBUF)]
            + [pltpu.VMEM((WIN, dc), jnp.float32) for _ in range(NBUF)]
            + [pltpu.VMEM((ch, dc), jnp.float32),
               pltpu.VMEM((ch, dc), jnp.float32),
               pltpu.VMEM((ch, dc), jnp.float32),
               pltpu.VMEM_SHARED((n, dc), jnp.float32)]
            + [pltpu.SemaphoreType.DMA for _ in range(3 * NBUF + 3)]
        ),
    )
    return k(hp3, src, dst, zrows_h)


def _pre_body(degp, d1o, d2o):
    dp = degp[...]
    d1o[...] = lax.rsqrt(dp[0, 0] + dp[1, 0] + 1.0)
    d2o[...] = lax.rsqrt(dp[0, 1] + dp[1, 1] + 1.0)


def _tc_pre(degp):
    n = degp.shape[2]
    return pl.pallas_call(
        _pre_body,
        out_shape=[
            jax.ShapeDtypeStruct((n, 1), jnp.float32),
            jax.ShapeDtypeStruct((n, 1), jnp.float32),
        ],
    )(degp)


def _bdot(a, b):
    return jnp.dot(a.astype(jnp.bfloat16), b.astype(jnp.bfloat16),
                   preferred_element_type=jnp.float32)


def _mm_body(act, w, dinv, out):
    h = _bdot(act[...], w[...])
    out[...] = (h * dinv[...])[None]


def _tc_mm(act, w, dinv):
    n, kdim = act.shape
    d = w.shape[1]
    nb = d // DC
    return pl.pallas_call(
        _mm_body,
        grid=(nb,),
        in_specs=[
            pl.BlockSpec((n, kdim), lambda b: (0, 0)),
            pl.BlockSpec((kdim, DC), lambda b: (0, b)),
            pl.BlockSpec((n, 1), lambda b: (0, 0)),
        ],
        out_specs=pl.BlockSpec((1, n, DC), lambda b: (b, 0, 0)),
        out_shape=jax.ShapeDtypeStruct((nb, n, DC), jnp.float32),
    )(act, w, dinv)


def _post_body(part, hp, dinv, bias, g, bt, out):
    p = part[...]
    dv = dinv[...]
    z = (p[0, 0] + p[1, 0] + hp[...][0]) * dv + bias[...]
    m = jnp.mean(z, axis=0, keepdims=True)
    v = jnp.mean((z - m) ** 2, axis=0, keepdims=True)
    out[...] = jnp.maximum(
        g[...] * (z - m) * lax.rsqrt(v + 1e-5) + bt[...], 0.0)


def _tc_post(part, hp3, dinv, bias, g, bt):
    _, nb, n, dc = part.shape
    d = nb * dc
    return pl.pallas_call(
        _post_body,
        grid=(nb,),
        in_specs=[
            pl.BlockSpec((2, 1, n, dc), lambda b: (0, b, 0, 0)),
            pl.BlockSpec((1, n, dc), lambda b: (b, 0, 0)),
            pl.BlockSpec((n, 1), lambda b: (0, 0)),
            pl.BlockSpec((1, dc), lambda b: (0, b)),
            pl.BlockSpec((1, dc), lambda b: (0, b)),
            pl.BlockSpec((1, dc), lambda b: (0, b)),
        ],
        out_specs=pl.BlockSpec((n, dc), lambda b: (0, b)),
        out_shape=jax.ShapeDtypeStruct((n, d), jnp.float32),
    )(part, hp3, dinv, bias.reshape(1, d), g.reshape(1, d), bt.reshape(1, d))


def _pool_body(act, batch, f1w, f1b, f2w, f2b, out, pooled):
    a = act[...]
    bt = batch[...]
    ng = pooled.shape[0]

    def body(b, carry):
        mask = (bt == b).astype(jnp.float32)
        pooled[pl.ds(b, 1), :] = jnp.max(a * mask, axis=0, keepdims=True)
        return carry

    lax.fori_loop(0, ng, body, 0)
    p = pooled[...]
    f = jnp.maximum(_bdot(p, f1w[...]) + f1b[...], 0.0)
    out[...] = _bdot(f, f2w[...]) + f2b[...]


def _tc_pool(act, batch2d, f1w, f1b, f2w, f2b, ng):
    d = act.shape[1]
    dmid = f1w.shape[1]
    dout = f2w.shape[1]
    return pl.pallas_call(
        _pool_body,
        out_shape=jax.ShapeDtypeStruct((ng, dout), jnp.float32),
        scratch_shapes=[pltpu.VMEM((ng, d), jnp.float32)],
    )(act, batch2d, f1w, f1b.reshape(1, dmid), f2w, f2b.reshape(1, dout))


def _head_body(xd1, xd2, cell, cw, cb, m1w, m1b, m2w, m2b, m3w, m3b, out):
    cl = jnp.maximum(_bdot(cell[...], cw[...]) + cb[...], 0.0)
    w = m1w[...]
    h = (_bdot(xd1[...], w[0:256]) + _bdot(xd2[...], w[256:512])
         + _bdot(cl, w[512:768]) + m1b[...])
    h = jnp.maximum(h, 0.0)
    h = jnp.maximum(_bdot(h, m2w[...]) + m2b[...], 0.0)
    o = _bdot(h, m3w[...]) + m3b[...]
    out[...] = 200.0 / (1.0 + jnp.exp(-o)) - 100.0


def _tc_head(xd1, xd2, cell, p):
    b = cell.shape[0]
    return pl.pallas_call(
        _head_body,
        out_shape=jax.ShapeDtypeStruct((b, 1), jnp.float32),
    )(xd1, xd2, cell, p["c_W"], p["c_b"].reshape(1, -1),
      p["m1W"], p["m1b"].reshape(1, -1), p["m2W"], p["m2b"].reshape(1, -1),
      p["m3W"], p["m3b"].reshape(1, -1))


def _gcn_branch(x, dinv, src, dst, batch2d, p, pre, ng):
    a = x
    for l in (1, 2, 3):
        hp3 = _tc_mm(a, p[pre + f"W{l}"], dinv)
        part = _sc_agg(hp3, src, dst)
        a = _tc_post(part, hp3, dinv, p[pre + f"b{l}"],
                     p[pre + f"g{l}"], p[pre + f"bt{l}"])
    return _tc_pool(a, batch2d, p[pre + "f1W"], p[pre + "f1b"],
                    p[pre + "f2W"], p[pre + "f2b"], ng)


def kernel(x1, edge_index1, x2, edge_index2, cellline, batch, params):
    p = params
    n = x1.shape[0]
    ng = cellline.shape[0]
    src1, dst1 = edge_index1[0], edge_index1[1]
    src2, dst2 = edge_index2[0], edge_index2[1]
    degp = _sc_deg(dst1, dst2, n).reshape(NC, 2, n, 1)
    dinv1, dinv2 = _tc_pre(degp)
    batch2d = batch.reshape(n, 1)
    xd1 = _gcn_branch(x1, dinv1, src1, dst1, batch2d, p, "d1_", ng)
    xd2 = _gcn_branch(x2, dinv2, src2, dst2, batch2d, p, "d2_", ng)
    return _tc_head(xd1, xd2, cellline, p)
